# baseline (speedup 1.0000x reference)
import functools

import jax
import jax.numpy as jnp
from jax import lax
from jax.experimental import pallas as pl
from jax.experimental.pallas import tpu as pltpu
from jax.experimental.pallas import tpu_sc as plsc


def _sc_gather(user, book, user_table, book_table):
    B = user.shape[0]
    E = user_table.shape[1]
    info = plsc.get_sparse_core_info()
    NW = info.num_cores * info.num_subcores
    bpw = B // NW
    mesh = plsc.VectorSubcoreMesh(core_axis_name="c", subcore_axis_name="s")

    @functools.partial(
        pl.kernel,
        mesh=mesh,
        compiler_params=pltpu.CompilerParams(use_tc_tiling_on_sc=False),
        out_type=(
            jax.ShapeDtypeStruct((B, E), jnp.float32),
            jax.ShapeDtypeStruct((B, E), jnp.float32),
        ),
        scratch_types=[
            pltpu.VMEM((bpw,), jnp.int32),
            pltpu.VMEM((bpw, E), jnp.float32),
            pltpu.VMEM((bpw,), jnp.int32),
            pltpu.VMEM((bpw, E), jnp.float32),
            pltpu.SemaphoreType.DMA,
        ],
    )
    def gk(user_hbm, book_hbm, ut_hbm, bt_hbm, ue_out, be_out,
           uidx, urows, bidx, brows, sem):
        wid = lax.axis_index("s") * info.num_cores + lax.axis_index("c")
        base = wid * bpw
        pltpu.sync_copy(user_hbm.at[pl.ds(base, bpw)], uidx)
        pltpu.sync_copy(book_hbm.at[pl.ds(base, bpw)], bidx)
        cu = pltpu.async_copy(ut_hbm.at[uidx], urows, sem)
        cb = pltpu.async_copy(bt_hbm.at[bidx], brows, sem)
        cu.wait()
        cb.wait()
        pltpu.sync_copy(urows, ue_out.at[pl.ds(base, bpw)])
        pltpu.sync_copy(brows, be_out.at[pl.ds(base, bpw)])

    return gk(user, book, user_table, book_table)


def _mlp_body(ue, be, w1u, w1b, b1, w2, b2, wo, bo, out):
    h = jnp.dot(ue[...], w1u[...], preferred_element_type=jnp.float32)
    h = h + jnp.dot(be[...], w1b[...], preferred_element_type=jnp.float32)
    h = jnp.maximum(h + b1[...], 0.0)
    h = jnp.dot(h, w2[...], preferred_element_type=jnp.float32)
    h = jnp.maximum(h + b2[...], 0.0)
    o = jnp.sum(h * wo[...], axis=1, keepdims=True) + bo[...]
    out[...] = jax.nn.sigmoid(o)


def _tc_mlp(ue, be, W1, b1, W2, b2, Wout, bout, blk=2048):
    B, E = ue.shape
    H1 = W1.shape[0]
    H2 = W2.shape[0]
    w1u = W1[:, :E].T
    w1b = W1[:, E:].T
    w2 = W2.T
    full = lambda shape: pl.BlockSpec(shape, lambda i: (0, 0))
    return pl.pallas_call(
        _mlp_body,
        grid=(B // blk,),
        in_specs=[
            pl.BlockSpec((blk, E), lambda i: (i, 0)),
            pl.BlockSpec((blk, E), lambda i: (i, 0)),
            full((E, H1)),
            full((E, H1)),
            full((1, H1)),
            full((H1, H2)),
            full((1, H2)),
            full((1, H2)),
            full((1, 1)),
        ],
        out_specs=pl.BlockSpec((blk, 1), lambda i: (i, 0)),
        out_shape=jax.ShapeDtypeStruct((B, 1), jnp.float32),
    )(ue, be, w1u, w1b, b1.reshape(1, H1), w2, b2.reshape(1, H2),
      Wout, bout.reshape(1, 1))


def kernel(user, book, user_table, book_table, W1, b1, W2, b2, Wout, bout):
    user = user.astype(jnp.int32)
    book = book.astype(jnp.int32)
    ue, be = _sc_gather(user, book, user_table, book_table)
    return _tc_mlp(ue, be, W1, b1, W2, b2, Wout, bout)
